# Initial kernel scaffold; baseline (speedup 1.0000x reference)
import jax
import jax.numpy as jnp
from jax.experimental import pallas as pl


def kernel(x, ones):
    raise NotImplementedError("write your pallas kernel here")


# speedup vs baseline: 1.1657x; 1.1657x over previous
import jax
import jax.numpy as jnp
from jax import lax
from jax.experimental import pallas as pl
from jax.experimental.pallas import tpu as pltpu
from jax.experimental.pallas import tpu_sc as plsc

_N = 1024 * 26
_D = 1000
_NC = 2
_NS = 16
_NW = _NC * _NS
_RPW = _N // _NW
_BLK = 32
_NBLK = _RPW // _BLK
_L = 16


def _sc_body(idx_hbm, out_hbm, idx_v, buf):
    wid = lax.axis_index("s") * _NC + lax.axis_index("c")
    base = wid * _RPW
    pltpu.sync_copy(idx_hbm.at[pl.ds(base, _RPW)], idx_v)

    lanes = lax.iota(jnp.int32, _L)
    zeros16 = jnp.zeros((_L,), jnp.float32)
    ones16 = jnp.ones((_L,), jnp.float32)

    def zinit(i, c):
        buf[pl.ds(i * _L, _L)] = zeros16
        return c

    lax.fori_loop(0, _BLK * _D // _L, zinit, 0)

    def step(b, c):
        row0 = b * _BLK
        for g in range(_BLK // _L):
            idxs = idx_v[pl.ds(row0 + g * _L, _L)]
            pos = (g * _L + lanes) * _D + idxs
            plsc.store_scatter(buf, [pos], ones16)
        pltpu.sync_copy(buf, out_hbm.at[pl.ds((base + row0) * _D, _BLK * _D)])
        for g in range(_BLK // _L):
            idxs = idx_v[pl.ds(row0 + g * _L, _L)]
            pos = (g * _L + lanes) * _D + idxs
            plsc.store_scatter(buf, [pos], zeros16)
        return c

    lax.fori_loop(0, _NBLK, step, 0)


def _one_hot_flat(flat_idx):
    mesh = plsc.VectorSubcoreMesh(core_axis_name="c", subcore_axis_name="s")
    f = pl.kernel(
        _sc_body,
        out_type=jax.ShapeDtypeStruct((_N * _D,), jnp.float32),
        mesh=mesh,
        scratch_types=[
            pltpu.VMEM((_RPW,), jnp.int32),
            pltpu.VMEM((_BLK * _D,), jnp.float32),
        ],
        compiler_params=pltpu.CompilerParams(needs_layout_passes=False),
    )
    return f(flat_idx)


def kernel(x, ones):
    depth = ones.shape[0]
    flat = x.reshape(-1).astype(jnp.int32)
    out = _one_hot_flat(flat)
    return out.reshape(x.shape + (depth,))
